# baseline (speedup 1.0000x reference)
import functools

import jax
import jax.numpy as jnp
import numpy as np
from jax.experimental import pallas as pl
from jax.experimental.pallas import tpu as pltpu

_TEMPERATURE = 0.02
_K = 10
_PRE = 5
_NEG = float(np.finfo(np.float32).min)


def _fused_body(row_ref, col_ref, unc_ref, ent_ref, tr_ref, tc_ref,
                *, k: int, max_ent: float):
    X = row_ref[...]
    Y = col_ref[...]
    blk = X.shape[0]
    B = X.shape[1]
    nch = B // 128

    cm = X.reshape(blk, nch, 128)
    rcands = []
    for _ in range(_PRE):
        t4 = jnp.max(cm, axis=2, keepdims=True)
        cm = jnp.where(cm >= t4, _NEG, cm)
        rcands.append(t4.reshape(blk, nch))
    rowmax = jnp.max(rcands[0], axis=1, keepdims=True)
    rcand = jnp.concatenate(rcands, axis=1)
    trc = None
    for _ in range(k):
        trc = jnp.max(rcand, axis=1, keepdims=True)
        rcand = jnp.where(rcand >= trc, _NEG, rcand)
    rcnt = jnp.sum((X >= trc).astype(jnp.float32), axis=1, keepdims=True)
    tr_ref[...] = trc
    bad_r = jnp.any(rcnt != float(k))

    @pl.when(bad_r)
    def _row_fallback():
        xm = X
        t = None
        for _ in range(k):
            t = jnp.max(xm, axis=1, keepdims=True)
            xm = jnp.where(xm >= t, _NEG, xm)
        tr_ref[...] = t

    ym = Y.reshape(nch, 128, blk)
    ccands = []
    for _ in range(_PRE):
        t4 = jnp.max(ym, axis=1, keepdims=True)
        ym = jnp.where(ym >= t4, _NEG, ym)
        ccands.append(t4.reshape(nch, blk))
    ccand = jnp.concatenate(ccands, axis=0)
    tcc = None
    for _ in range(k):
        tcc = jnp.max(ccand, axis=0, keepdims=True)
        ccand = jnp.where(ccand >= tcc, _NEG, ccand)
    ccnt = jnp.sum((Y >= tcc).astype(jnp.float32), axis=0, keepdims=True)
    tc_ref[...] = tcc
    bad_c = jnp.any(ccnt != float(k))

    @pl.when(bad_c)
    def _col_fallback():
        ym2 = Y
        t = None
        for _ in range(k):
            t = jnp.max(ym2, axis=0, keepdims=True)
            ym2 = jnp.where(ym2 >= t, _NEG, ym2)
        tc_ref[...] = t

    tr = tr_ref[...]
    tc = tc_ref[...]

    inv_t = 1.0 / _TEMPERATURE
    sm = (X - rowmax) * inv_t
    e = jnp.exp(sm)
    Z = jnp.sum(e, axis=1, keepdims=True)
    S1 = jnp.sum(sm * e, axis=1, keepdims=True)
    ent = (jnp.log(Z) - S1 / Z)[:, 0] * (1.0 / max_ent)

    R = (X >= tr).astype(jnp.float32)
    C = (Y >= tc).astype(jnp.float32)
    P = jax.lax.dot(R, C, preferred_element_type=jnp.float32)
    ii = jax.lax.broadcasted_iota(jnp.int32, (blk, blk), 0)
    jj = jax.lax.broadcasted_iota(jnp.int32, (blk, blk), 1)
    matches = jnp.sum(jnp.where(ii == jj, P, 0.0), axis=1)

    ra = matches * (1.0 / k)
    unc_ref[...] = (1.0 - ra) * 0.5 + ent * 0.5
    ent_ref[...] = ent


def kernel(sim_matrix, pids):
    del pids
    B = sim_matrix.shape[0]
    blk = 256
    k = min(_K, B)
    max_ent = float(np.log(B + 1e-10))
    grid = B // blk
    unc, ent = pl.pallas_call(
        functools.partial(_fused_body, k=k, max_ent=max_ent),
        grid=(grid,),
        in_specs=[
            pl.BlockSpec((blk, B), lambda i: (i, 0)),
            pl.BlockSpec((B, blk), lambda i: (0, i)),
        ],
        out_specs=[
            pl.BlockSpec((blk,), lambda i: (i,)),
            pl.BlockSpec((blk,), lambda i: (i,)),
        ],
        out_shape=[
            jax.ShapeDtypeStruct((B,), jnp.float32),
            jax.ShapeDtypeStruct((B,), jnp.float32),
        ],
        scratch_shapes=[
            pltpu.VMEM((blk, 1), jnp.float32),
            pltpu.VMEM((1, blk), jnp.float32),
        ],
    )(sim_matrix, sim_matrix)
    return (unc, ent)


# speedup vs baseline: 1.4448x; 1.4448x over previous
import functools

import jax
import jax.numpy as jnp
import numpy as np
from jax.experimental import pallas as pl
from jax.experimental.pallas import tpu as pltpu

_TEMPERATURE = 0.02
_K = 10
_NEG = float(np.finfo(np.float32).min)


def _fused_body(row_ref, col_ref, unc_ref, ent_ref, *, k: int, max_ent: float):
    X = row_ref[...]
    Y = col_ref[...]
    blk = X.shape[0]

    xm = X
    tr = None
    rowmax = None
    for it in range(k):
        tr = jnp.max(xm, axis=1, keepdims=True)
        if it == 0:
            rowmax = tr
        xm = jnp.where(xm >= tr, _NEG, xm)

    ym = Y
    tc = None
    for _ in range(k):
        tc = jnp.max(ym, axis=0, keepdims=True)
        ym = jnp.where(ym >= tc, _NEG, ym)

    inv_t = 1.0 / _TEMPERATURE
    sm = (X - rowmax) * inv_t
    e = jnp.exp(sm)
    Z = jnp.sum(e, axis=1, keepdims=True)
    S1 = jnp.sum(sm * e, axis=1, keepdims=True)
    ent = (jnp.log(Z) - S1 / Z)[:, 0] * (1.0 / max_ent)

    R = (X >= tr).astype(jnp.float32)
    C = (Y >= tc).astype(jnp.float32)
    P = jax.lax.dot(R, C, preferred_element_type=jnp.float32)
    ii = jax.lax.broadcasted_iota(jnp.int32, (blk, blk), 0)
    jj = jax.lax.broadcasted_iota(jnp.int32, (blk, blk), 1)
    matches = jnp.sum(jnp.where(ii == jj, P, 0.0), axis=1)

    ra = matches * (1.0 / k)
    unc_ref[...] = (1.0 - ra) * 0.5 + ent * 0.5
    ent_ref[...] = ent


def kernel(sim_matrix, pids):
    del pids
    B = sim_matrix.shape[0]
    blk = 512
    k = min(_K, B)
    max_ent = float(np.log(B + 1e-10))
    grid = B // blk
    unc, ent = pl.pallas_call(
        functools.partial(_fused_body, k=k, max_ent=max_ent),
        grid=(grid,),
        in_specs=[
            pl.BlockSpec((blk, B), lambda i: (i, 0)),
            pl.BlockSpec((B, blk), lambda i: (0, i)),
        ],
        out_specs=[
            pl.BlockSpec((blk,), lambda i: (i,)),
            pl.BlockSpec((blk,), lambda i: (i,)),
        ],
        out_shape=[
            jax.ShapeDtypeStruct((B,), jnp.float32),
            jax.ShapeDtypeStruct((B,), jnp.float32),
        ],
    )(sim_matrix, sim_matrix)
    return (unc, ent)


# speedup vs baseline: 1.4575x; 1.0088x over previous
import functools

import jax
import jax.numpy as jnp
import numpy as np
from jax.experimental import pallas as pl
from jax.experimental.pallas import tpu as pltpu

_TEMPERATURE = 0.02
_K = 10
_NEG = float(np.finfo(np.float32).min)


def _fused_body(row_ref, col_ref, unc_ref, ent_ref, *, k: int, max_ent: float):
    X = row_ref[...]
    Y = col_ref[...]
    blk = X.shape[0]

    tr = jnp.max(X, axis=1, keepdims=True)
    rowmax = tr
    for _ in range(k - 1):
        tr = jnp.max(jnp.where(X >= tr, _NEG, X), axis=1, keepdims=True)

    tc = jnp.max(Y, axis=0, keepdims=True)
    for _ in range(k - 1):
        tc = jnp.max(jnp.where(Y >= tc, _NEG, Y), axis=0, keepdims=True)

    inv_t = 1.0 / _TEMPERATURE
    sm = (X - rowmax) * inv_t
    e = jnp.exp(sm)
    Z = jnp.sum(e, axis=1, keepdims=True)
    S1 = jnp.sum(sm * e, axis=1, keepdims=True)
    ent = (jnp.log(Z) - S1 / Z)[:, 0] * (1.0 / max_ent)

    R = (X >= tr).astype(jnp.float32)
    C = (Y >= tc).astype(jnp.float32)
    P = jax.lax.dot(R, C, preferred_element_type=jnp.float32)
    ii = jax.lax.broadcasted_iota(jnp.int32, (blk, blk), 0)
    jj = jax.lax.broadcasted_iota(jnp.int32, (blk, blk), 1)
    matches = jnp.sum(jnp.where(ii == jj, P, 0.0), axis=1)

    ra = matches * (1.0 / k)
    unc_ref[...] = (1.0 - ra) * 0.5 + ent * 0.5
    ent_ref[...] = ent


def kernel(sim_matrix, pids):
    del pids
    B = sim_matrix.shape[0]
    blk = 512
    k = min(_K, B)
    max_ent = float(np.log(B + 1e-10))
    grid = B // blk
    unc, ent = pl.pallas_call(
        functools.partial(_fused_body, k=k, max_ent=max_ent),
        grid=(grid,),
        in_specs=[
            pl.BlockSpec((blk, B), lambda i: (i, 0)),
            pl.BlockSpec((B, blk), lambda i: (0, i)),
        ],
        out_specs=[
            pl.BlockSpec((blk,), lambda i: (i,)),
            pl.BlockSpec((blk,), lambda i: (i,)),
        ],
        out_shape=[
            jax.ShapeDtypeStruct((B,), jnp.float32),
            jax.ShapeDtypeStruct((B,), jnp.float32),
        ],
    )(sim_matrix, sim_matrix)
    return (unc, ent)
